# Initial kernel scaffold; baseline (speedup 1.0000x reference)
import jax
import jax.numpy as jnp
from jax.experimental import pallas as pl


def kernel(atom_ft, bond_ft, edge_index):
    raise NotImplementedError("write your pallas kernel here")


# speedup vs baseline: 2.5548x; 2.5548x over previous
import functools

import jax
import jax.numpy as jnp
from jax import lax
from jax.experimental import pallas as pl
from jax.experimental.pallas import tpu as pltpu
from jax.experimental.pallas import tpu_sc as plsc

N_ATOM = 10000
N_BOND = 10000
K = 32
D = 128

NTOT = (K + 1) * N_BOND
G = 120
NG = NTOT // G

_info = plsc.get_sparse_core_info()
NC, NS = _info.num_cores, _info.num_subcores
NW = NC * NS
ITERS = -(-NG // NW)


@functools.partial(
    pl.kernel,
    mesh=plsc.VectorSubcoreMesh(core_axis_name="c", subcore_axis_name="s"),
    out_type=jax.ShapeDtypeStruct((NTOT, D), jnp.float32),
    scratch_types=[
        pltpu.VMEM((G,), jnp.int32),
        pltpu.VMEM((G, D), jnp.float32),
        pltpu.SemaphoreType.DMA,
    ],
)
def _gather_rows(table_hbm, idx_hbm, out_hbm, idx_v, rows_v, sem):
    wid = lax.axis_index("s") * NC + lax.axis_index("c")

    def body(i, carry):
        c = i * NW + wid

        @pl.when(c < NG)
        def _():
            pltpu.sync_copy(idx_hbm.at[c], idx_v)
            pltpu.async_copy(table_hbm.at[idx_v], rows_v, sem).wait()
            pltpu.sync_copy(rows_v, out_hbm.at[pl.ds(c * G, G)])

        return carry

    lax.fori_loop(0, ITERS, body, 0)


def kernel(atom_ft, bond_ft, edge_index):
    src = edge_index[0]
    table = jnp.concatenate([atom_ft, bond_ft], axis=0)
    bond_ids = jnp.arange(N_BOND, dtype=jnp.int32) + N_ATOM
    full_idx = jnp.concatenate(
        [bond_ids[:, None], src.reshape(N_BOND, K)], axis=1
    ).reshape(NG, G)
    out = _gather_rows(table, full_idx)
    return (atom_ft, out.reshape(N_BOND, (K + 1) * D))


# speedup vs baseline: 2.9775x; 1.1654x over previous
import functools

import jax
import jax.numpy as jnp
from jax import lax
from jax.experimental import pallas as pl
from jax.experimental.pallas import tpu as pltpu
from jax.experimental.pallas import tpu_sc as plsc

N_ATOM = 10000
N_BOND = 10000
K = 32
D = 128

NTOT = (K + 1) * N_BOND
G = 120
NG = NTOT // G

_info = plsc.get_sparse_core_info()
NC, NS = _info.num_cores, _info.num_subcores
NW = NC * NS
CPW = ((-(-NG // NW)) + 7) // 8 * 8
NGP = CPW * NW


@functools.partial(
    pl.kernel,
    mesh=plsc.VectorSubcoreMesh(core_axis_name="c", subcore_axis_name="s"),
    out_type=jax.ShapeDtypeStruct((NTOT, D), jnp.float32),
    scratch_types=[
        pltpu.VMEM((CPW, G), jnp.int32),
        pltpu.VMEM((2, G, D), jnp.float32),
        pltpu.SemaphoreType.DMA,
        pltpu.SemaphoreType.DMA,
        pltpu.SemaphoreType.DMA,
        pltpu.SemaphoreType.DMA,
    ],
)
def _gather_rows(table_hbm, idx_hbm, out_hbm, idx_v, rows_v, g0, g1, s0, s1):
    wid = lax.axis_index("s") * NC + lax.axis_index("c")
    c0 = wid * CPW
    pltpu.sync_copy(idx_hbm.at[pl.ds(c0, CPW)], idx_v)

    gs = (g0, g1)
    ss = (s0, s1)

    def g_copy(j, b):
        return pltpu.make_async_copy(
            table_hbm.at[idx_v.at[j]], rows_v.at[b], gs[b])

    def s_copy(j, b):
        return pltpu.make_async_copy(
            rows_v.at[b], out_hbm.at[pl.ds((c0 + j) * G, G)], ss[b])

    def live(j):
        return (j >= 0) & (j < CPW) & (c0 + j < NG)

    def start(j, b):
        @pl.when(live(j))
        def _():
            g_copy(j, b).start()

    def finish(j, b):
        @pl.when(live(j))
        def _():
            g_copy(j, b).wait()
            s_copy(j, b).start()

    def drain(j, b):
        @pl.when(live(j))
        def _():
            s_copy(j, b).wait()

    start(0, 0)

    def body(p, carry):
        j0 = 2 * p
        j1 = j0 + 1
        finish(j0, 0)
        drain(j1 - 2, 1)
        start(j1, 1)
        finish(j1, 1)
        drain(j0, 0)
        start(j0 + 2, 0)
        return carry

    lax.fori_loop(0, CPW // 2, body, 0)
    drain(CPW - 1, 1)


def kernel(atom_ft, bond_ft, edge_index):
    src = edge_index[0]
    table = jnp.concatenate([atom_ft, bond_ft], axis=0)
    bond_ids = jnp.arange(N_BOND, dtype=jnp.int32) + N_ATOM
    full_idx = jnp.concatenate(
        [bond_ids[:, None], src.reshape(N_BOND, K)], axis=1
    ).reshape(-1)
    full_idx = jnp.concatenate(
        [full_idx, jnp.zeros(NGP * G - NTOT, jnp.int32)]
    ).reshape(NGP, G)
    out = _gather_rows(table, full_idx)
    return (atom_ft, out.reshape(N_BOND, (K + 1) * D))


# speedup vs baseline: 5.1393x; 1.7261x over previous
import functools

import jax
import jax.numpy as jnp
from jax import lax
from jax.experimental import pallas as pl
from jax.experimental.pallas import tpu as pltpu
from jax.experimental.pallas import tpu_sc as plsc

N_ATOM = 10000
N_BOND = 10000
K = 32
D = 128

NB = 80
NBLK = N_BOND // NB
NCH = NBLK * K

_info = plsc.get_sparse_core_info()
NC, NS = _info.num_cores, _info.num_subcores
NW = NC * NS
BPW = -(-NBLK // NW)
CPW = BPW * K
NCHP = CPW * NW


@functools.partial(
    pl.kernel,
    mesh=plsc.VectorSubcoreMesh(core_axis_name="c", subcore_axis_name="s"),
    out_type=jax.ShapeDtypeStruct((N_BOND, (K + 1) * D), jnp.float32),
    scratch_types=[
        pltpu.VMEM((CPW, NB), jnp.int32),
        pltpu.VMEM((2, NB, D), jnp.float32),
        pltpu.SemaphoreType.DMA,
        pltpu.SemaphoreType.DMA,
        pltpu.SemaphoreType.DMA,
        pltpu.SemaphoreType.DMA,
        pltpu.SemaphoreType.DMA,
    ],
)
def _concat_pool(atom_hbm, bond_hbm, idx_hbm, out_hbm, idx_v, rows_v,
                 g0, g1, s0, s1, bsem):
    wid = lax.axis_index("s") * NC + lax.axis_index("c")
    ch0 = wid * CPW
    blk0 = wid * BPW
    pltpu.sync_copy(idx_hbm.at[pl.ds(ch0, CPW)], idx_v)

    def b_copy(i):
        blk = blk0 + i
        r0 = pl.multiple_of(blk * NB, NB)
        return pltpu.make_async_copy(
            bond_hbm.at[pl.ds(r0, NB)],
            out_hbm.at[pl.ds(r0, NB), pl.ds(0, D)], bsem)

    for i in range(BPW):
        @pl.when(blk0 + i < NBLK)
        def _():
            b_copy(i).start()

    gs = (g0, g1)
    ss = (s0, s1)

    def g_copy(j, b):
        return pltpu.make_async_copy(
            atom_hbm.at[idx_v.at[j]], rows_v.at[b], gs[b])

    def s_copy(j, b):
        c = ch0 + j
        blk = c // K
        t = c % K
        r0 = pl.multiple_of(blk * NB, NB)
        c0 = pl.multiple_of((t + 1) * D, D)
        return pltpu.make_async_copy(
            rows_v.at[b], out_hbm.at[pl.ds(r0, NB), pl.ds(c0, D)], ss[b])

    def live(j):
        return (j >= 0) & (j < CPW) & (ch0 + j < NCH)

    def start(j, b):
        @pl.when(live(j))
        def _():
            g_copy(j, b).start()

    def finish(j, b):
        @pl.when(live(j))
        def _():
            g_copy(j, b).wait()
            s_copy(j, b).start()

    def drain(j, b):
        @pl.when(live(j))
        def _():
            s_copy(j, b).wait()

    start(0, 0)

    def body(p, carry):
        j0 = 2 * p
        j1 = j0 + 1
        finish(j0, 0)
        drain(j1 - 2, 1)
        start(j1, 1)
        finish(j1, 1)
        drain(j0, 0)
        start(j0 + 2, 0)
        return carry

    lax.fori_loop(0, CPW // 2, body, 0)
    drain(CPW - 1, 1)

    for i in range(BPW):
        @pl.when(blk0 + i < NBLK)
        def _():
            b_copy(i).wait()


def kernel(atom_ft, bond_ft, edge_index):
    src = edge_index[0]
    idx = src.reshape(NBLK, NB, K).transpose(0, 2, 1).reshape(NCH, NB)
    idx = jnp.concatenate(
        [idx, jnp.zeros((NCHP - NCH, NB), jnp.int32)], axis=0)
    out = _concat_pool(atom_ft, bond_ft, idx)
    return (atom_ft, out)


# speedup vs baseline: 5.7603x; 1.1208x over previous
import functools

import jax
import jax.numpy as jnp
from jax import lax
from jax.experimental import pallas as pl
from jax.experimental.pallas import tpu as pltpu
from jax.experimental.pallas import tpu_sc as plsc

N_ATOM = 10000
N_BOND = 10000
K = 32
D = 128

NB = 80
NBLK = N_BOND // NB
NCH = NBLK * K

_info = plsc.get_sparse_core_info()
NC, NS = _info.num_cores, _info.num_subcores
NW = NC * NS
BPW = -(-NBLK // NW)
CPW = BPW * K
NCHP = CPW * NW


@functools.partial(
    pl.kernel,
    mesh=plsc.VectorSubcoreMesh(core_axis_name="c", subcore_axis_name="s"),
    out_type=jax.ShapeDtypeStruct((N_BOND, (K + 1) * D), jnp.float32),
    scratch_types=[
        pltpu.VMEM((CPW, NB), jnp.int32),
        pltpu.VMEM((4, NB, D), jnp.float32),
        pltpu.SemaphoreType.DMA,
        pltpu.SemaphoreType.DMA,
        pltpu.SemaphoreType.DMA,
        pltpu.SemaphoreType.DMA,
        pltpu.SemaphoreType.DMA,
        pltpu.SemaphoreType.DMA,
        pltpu.SemaphoreType.DMA,
        pltpu.SemaphoreType.DMA,
        pltpu.SemaphoreType.DMA,
    ],
)
def _concat_pool(atom_hbm, bond_hbm, idx_hbm, out_hbm, idx_v, rows_v,
                 g0, g1, g2, g3, s0, s1, s2, s3, bsem):
    wid = lax.axis_index("s") * NC + lax.axis_index("c")
    ch0 = wid * CPW
    blk0 = wid * BPW
    pltpu.sync_copy(idx_hbm.at[pl.ds(ch0, CPW)], idx_v)

    def b_copy(i):
        blk = blk0 + i
        r0 = pl.multiple_of(blk * NB, NB)
        return pltpu.make_async_copy(
            bond_hbm.at[pl.ds(r0, NB)],
            out_hbm.at[pl.ds(r0, NB), pl.ds(0, D)], bsem)

    for i in range(BPW):
        @pl.when(blk0 + i < NBLK)
        def _():
            b_copy(i).start()

    gs = (g0, g1, g2, g3)
    ss = (s0, s1, s2, s3)

    def g_copy(j, b):
        return pltpu.make_async_copy(
            atom_hbm.at[idx_v.at[j]], rows_v.at[b], gs[b])

    def s_copy(j, b):
        c = ch0 + j
        blk = c // K
        t = c % K
        r0 = pl.multiple_of(blk * NB, NB)
        c0 = pl.multiple_of((t + 1) * D, D)
        return pltpu.make_async_copy(
            rows_v.at[b], out_hbm.at[pl.ds(r0, NB), pl.ds(c0, D)], ss[b])

    def live(j):
        return (j >= 0) & (j < CPW) & (ch0 + j < NCH)

    def start(j, b):
        @pl.when(live(j))
        def _():
            g_copy(j, b).start()

    def finish(j, b):
        @pl.when(live(j))
        def _():
            g_copy(j, b).wait()
            s_copy(j, b).start()

    def drain(j, b):
        @pl.when(live(j))
        def _():
            s_copy(j, b).wait()

    for b in range(4):
        start(b, b)

    def body(q, carry):
        j = 4 * q
        for b in range(4):
            finish(j + b, b)
        for b in range(4):
            drain(j + b, b)
            start(j + b + 4, b)
        return carry

    lax.fori_loop(0, CPW // 4, body, 0)

    for i in range(BPW):
        @pl.when(blk0 + i < NBLK)
        def _():
            b_copy(i).wait()


def kernel(atom_ft, bond_ft, edge_index):
    src = edge_index[0]
    idx = src.reshape(NBLK, NB, K).transpose(0, 2, 1).reshape(NCH, NB)
    idx = jnp.concatenate(
        [idx, jnp.zeros((NCHP - NCH, NB), jnp.int32)], axis=0)
    out = _concat_pool(atom_ft, bond_ft, idx)
    return (atom_ft, out)
